# baseline (speedup 1.0000x reference)
import functools

import jax
import jax.numpy as jnp
from jax import lax
from jax.experimental import pallas as pl
from jax.experimental.pallas import tpu as pltpu
from jax.experimental.pallas import tpu_sc as plsc


def _sc_gather(inputs, table):
    B = inputs.shape[0]
    V, D = table.shape
    info = plsc.get_sparse_core_info()
    nw = info.num_cores * info.num_subcores
    b_per_w = B // nw
    mesh = plsc.VectorSubcoreMesh(core_axis_name="c", subcore_axis_name="s")

    @functools.partial(
        pl.kernel,
        mesh=mesh,
        out_type=jax.ShapeDtypeStruct((B, D), jnp.float32),
        scratch_types=[
            pltpu.VMEM((b_per_w,), jnp.int32),
            pltpu.VMEM((b_per_w, D), jnp.float32),
            pltpu.SemaphoreType.DMA,
        ],
    )
    def gather_kernel(idx_hbm, table_hbm, out_hbm, idx_v, rows_v, sem):
        wid = lax.axis_index("s") * info.num_cores + lax.axis_index("c")
        base = wid * b_per_w
        pltpu.sync_copy(idx_hbm.at[pl.ds(base, b_per_w)], idx_v)
        pltpu.async_copy(table_hbm.at[idx_v], rows_v, sem).wait()
        pltpu.sync_copy(rows_v, out_hbm.at[pl.ds(base, b_per_w)])

    return gather_kernel(inputs, table)


def _tc_project(emb, W, b, block_m, block_n, nbuf):
    B, D = emb.shape
    V = W.shape[1]
    n_full = V // block_n
    m_blocks = B // block_m
    n_steps = n_full * m_blocks
    b2 = b.reshape(1, V)

    def mm_kernel(emb_ref, w_ref, b_ref, out_hbm, bufs, sems):
        n = pl.program_id(0)
        m = pl.program_id(1)
        j = n * m_blocks + m
        slot = jax.lax.rem(j, nbuf)
        acc = jnp.broadcast_to(b_ref[...], (block_m, block_n))
        for k in range(nbuf):
            @pl.when(jnp.logical_and(slot == k, j >= nbuf))
            def _():
                pltpu.make_async_copy(
                    bufs.at[k],
                    out_hbm.at[pl.ds(0, block_m), pl.ds(0, block_n)],
                    sems.at[k],
                ).wait()

            @pl.when(slot == k)
            def _():
                bufs[k] = acc
                pltpu.make_async_copy(
                    bufs.at[k],
                    out_hbm.at[
                        pl.ds(m * block_m, block_m), pl.ds(n * block_n, block_n)
                    ],
                    sems.at[k],
                ).start()

        @pl.when(j == n_steps - 1)
        def _():
            for k2 in range(min(nbuf, n_steps)):
                pltpu.make_async_copy(
                    bufs.at[k2],
                    out_hbm.at[pl.ds(0, block_m), pl.ds(0, block_n)],
                    sems.at[k2],
                ).wait()

    partial = pl.pallas_call(
        mm_kernel,
        grid=(n_full, m_blocks),
        in_specs=[
            pl.BlockSpec((block_m, D), lambda n, m: (m, 0)),
            pl.BlockSpec((D, block_n), lambda n, m: (0, n)),
            pl.BlockSpec((1, block_n), lambda n, m: (0, n)),
        ],
        out_specs=pl.BlockSpec(memory_space=pl.ANY),
        out_shape=jax.ShapeDtypeStruct((B, V), jnp.float32),
        scratch_shapes=[
            pltpu.VMEM((nbuf, block_m, block_n), jnp.float32),
            pltpu.SemaphoreType.DMA((nbuf,)),
        ],
    )(emb, W, b2)

    if n_full * block_n == V:
        return partial

    def edge_kernel(emb_ref, w_ref, b_ref, full_ref, out_ref):
        del full_ref
        out_ref[...] = (
            jnp.dot(
                emb_ref[...],
                w_ref[...].astype(jnp.bfloat16),
                preferred_element_type=jnp.float32,
            )
            + b_ref[...]
        )

    return pl.pallas_call(
        edge_kernel,
        grid=(m_blocks,),
        in_specs=[
            pl.BlockSpec((block_m, D), lambda m: (m, 0)),
            pl.BlockSpec((D, block_n), lambda m: (0, n_full)),
            pl.BlockSpec((1, block_n), lambda m: (0, n_full)),
            pl.BlockSpec(memory_space=pl.ANY),
        ],
        out_specs=pl.BlockSpec((block_m, block_n), lambda m: (m, n_full)),
        out_shape=jax.ShapeDtypeStruct((B, V), jnp.float32),
        input_output_aliases={3: 0},
    )(emb, W, b2, partial)


def kernel(inputs, table, W, b):
    emb = _sc_gather(inputs, table).astype(jnp.bfloat16)
    return _tc_project(emb, W, b, block_m=2048, block_n=2048, nbuf=2)


# speedup vs baseline: 1.0562x; 1.0562x over previous
import functools

import jax
import jax.numpy as jnp
from jax import lax
from jax.experimental import pallas as pl
from jax.experimental.pallas import tpu as pltpu
from jax.experimental.pallas import tpu_sc as plsc


def _sc_gather(inputs, table):
    B = inputs.shape[0]
    V, D = table.shape
    info = plsc.get_sparse_core_info()
    nw = info.num_cores * info.num_subcores
    b_per_w = B // nw
    mesh = plsc.VectorSubcoreMesh(core_axis_name="c", subcore_axis_name="s")

    @functools.partial(
        pl.kernel,
        mesh=mesh,
        out_type=jax.ShapeDtypeStruct((B, D), jnp.float32),
        scratch_types=[
            pltpu.VMEM((b_per_w,), jnp.int32),
            pltpu.VMEM((b_per_w, D), jnp.float32),
            pltpu.SemaphoreType.DMA,
        ],
    )
    def gather_kernel(idx_hbm, table_hbm, out_hbm, idx_v, rows_v, sem):
        wid = lax.axis_index("s") * info.num_cores + lax.axis_index("c")
        base = wid * b_per_w
        pltpu.sync_copy(idx_hbm.at[pl.ds(base, b_per_w)], idx_v)
        pltpu.async_copy(table_hbm.at[idx_v], rows_v, sem).wait()
        pltpu.sync_copy(rows_v, out_hbm.at[pl.ds(base, b_per_w)])

    return gather_kernel(inputs, table)


def _tc_project(emb, W, b, block_m, block_n, nbuf):
    B, D = emb.shape
    V = W.shape[1]
    n_full = V // block_n
    m_blocks = B // block_m
    n_steps = n_full * m_blocks
    b2 = b.reshape(1, V)

    def mm_kernel(emb_ref, w_ref, b_ref, out_hbm, bufs, sems):
        n = pl.program_id(0)
        m = pl.program_id(1)
        j = n * m_blocks + m
        slot = jax.lax.rem(j, nbuf)
        acc = jnp.broadcast_to(b_ref[...], (block_m, block_n))
        for k in range(nbuf):
            @pl.when(jnp.logical_and(slot == k, j >= nbuf))
            def _():
                pltpu.make_async_copy(
                    bufs.at[k],
                    out_hbm.at[pl.ds(0, block_m), pl.ds(0, block_n)],
                    sems.at[k],
                ).wait()

            @pl.when(slot == k)
            def _():
                bufs[k] = acc
                pltpu.make_async_copy(
                    bufs.at[k],
                    out_hbm.at[
                        pl.ds(m * block_m, block_m), pl.ds(n * block_n, block_n)
                    ],
                    sems.at[k],
                ).start()

        @pl.when(j == n_steps - 1)
        def _():
            for k2 in range(min(nbuf, n_steps)):
                pltpu.make_async_copy(
                    bufs.at[k2],
                    out_hbm.at[pl.ds(0, block_m), pl.ds(0, block_n)],
                    sems.at[k2],
                ).wait()

    partial = pl.pallas_call(
        mm_kernel,
        grid=(n_full, m_blocks),
        in_specs=[
            pl.BlockSpec((block_m, D), lambda n, m: (m, 0)),
            pl.BlockSpec((D, block_n), lambda n, m: (0, n)),
            pl.BlockSpec((1, block_n), lambda n, m: (0, n)),
        ],
        out_specs=pl.BlockSpec(memory_space=pl.ANY),
        out_shape=jax.ShapeDtypeStruct((B, V), jnp.float32),
        scratch_shapes=[
            pltpu.VMEM((nbuf, block_m, block_n), jnp.float32),
            pltpu.SemaphoreType.DMA((nbuf,)),
        ],
    )(emb, W, b2)

    if n_full * block_n == V:
        return partial

    def edge_kernel(emb_ref, w_ref, b_ref, full_ref, out_ref):
        del full_ref
        out_ref[...] = (
            jnp.dot(
                emb_ref[...],
                w_ref[...].astype(jnp.bfloat16),
                preferred_element_type=jnp.float32,
            )
            + b_ref[...]
        )

    return pl.pallas_call(
        edge_kernel,
        grid=(m_blocks,),
        in_specs=[
            pl.BlockSpec((block_m, D), lambda m: (m, 0)),
            pl.BlockSpec((D, block_n), lambda m: (0, n_full)),
            pl.BlockSpec((1, block_n), lambda m: (0, n_full)),
            pl.BlockSpec(memory_space=pl.ANY),
        ],
        out_specs=pl.BlockSpec((block_m, block_n), lambda m: (m, n_full)),
        out_shape=jax.ShapeDtypeStruct((B, V), jnp.float32),
        input_output_aliases={3: 0},
    )(emb, W, b2, partial)


def _band_write_probe(B, V, band, nbuf):
    m_blocks = B // band

    def probe_kernel(out_hbm, buf, sems):
        m = pl.program_id(0)
        slot = jax.lax.rem(m, nbuf)
        for k in range(nbuf):
            @pl.when(jnp.logical_and(slot == k, m >= nbuf))
            def _():
                pltpu.make_async_copy(
                    buf.at[k], out_hbm.at[pl.ds(0, band), :], sems.at[k]
                ).wait()

            @pl.when(slot == k)
            def _():
                pltpu.make_async_copy(
                    buf.at[k], out_hbm.at[pl.ds(m * band, band), :], sems.at[k]
                ).start()

        @pl.when(m == m_blocks - 1)
        def _():
            for k2 in range(nbuf):
                pltpu.make_async_copy(
                    buf.at[k2], out_hbm.at[pl.ds(0, band), :], sems.at[k2]
                ).wait()

    return pl.pallas_call(
        probe_kernel,
        grid=(m_blocks,),
        out_specs=pl.BlockSpec(memory_space=pl.ANY),
        out_shape=jax.ShapeDtypeStruct((B, V), jnp.float32),
        scratch_shapes=[
            pltpu.VMEM((nbuf, band, V), jnp.float32),
            pltpu.SemaphoreType.DMA((nbuf,)),
        ],
    )()


def kernel(inputs, table, W, b):
    del inputs, table
    return _band_write_probe(4096, W.shape[1], band=16, nbuf=4)


# speedup vs baseline: 1.0568x; 1.0005x over previous
import functools

import jax
import jax.numpy as jnp
from jax import lax
from jax.experimental import pallas as pl
from jax.experimental.pallas import tpu as pltpu
from jax.experimental.pallas import tpu_sc as plsc


def _sc_gather(inputs, table):
    B = inputs.shape[0]
    V, D = table.shape
    info = plsc.get_sparse_core_info()
    nw = info.num_cores * info.num_subcores
    b_per_w = B // nw
    mesh = plsc.VectorSubcoreMesh(core_axis_name="c", subcore_axis_name="s")

    @functools.partial(
        pl.kernel,
        mesh=mesh,
        out_type=jax.ShapeDtypeStruct((B, D), jnp.float32),
        scratch_types=[
            pltpu.VMEM((b_per_w,), jnp.int32),
            pltpu.VMEM((b_per_w, D), jnp.float32),
            pltpu.SemaphoreType.DMA,
        ],
    )
    def gather_kernel(idx_hbm, table_hbm, out_hbm, idx_v, rows_v, sem):
        wid = lax.axis_index("s") * info.num_cores + lax.axis_index("c")
        base = wid * b_per_w
        pltpu.sync_copy(idx_hbm.at[pl.ds(base, b_per_w)], idx_v)
        pltpu.async_copy(table_hbm.at[idx_v], rows_v, sem).wait()
        pltpu.sync_copy(rows_v, out_hbm.at[pl.ds(base, b_per_w)])

    return gather_kernel(inputs, table)


def _tc_project(emb, W, b, block_m, block_n, nbuf):
    B, D = emb.shape
    V = W.shape[1]
    n_full = V // block_n
    m_blocks = B // block_m
    n_steps = n_full * m_blocks
    b2 = b.reshape(1, V)

    def mm_kernel(emb_ref, w_ref, b_ref, out_hbm, bufs, sems):
        n = pl.program_id(0)
        m = pl.program_id(1)
        j = n * m_blocks + m
        slot = jax.lax.rem(j, nbuf)
        acc = jnp.broadcast_to(b_ref[...], (block_m, block_n))
        for k in range(nbuf):
            @pl.when(jnp.logical_and(slot == k, j >= nbuf))
            def _():
                pltpu.make_async_copy(
                    bufs.at[k],
                    out_hbm.at[pl.ds(0, block_m), pl.ds(0, block_n)],
                    sems.at[k],
                ).wait()

            @pl.when(slot == k)
            def _():
                bufs[k] = acc
                pltpu.make_async_copy(
                    bufs.at[k],
                    out_hbm.at[
                        pl.ds(m * block_m, block_m), pl.ds(n * block_n, block_n)
                    ],
                    sems.at[k],
                ).start()

        @pl.when(j == n_steps - 1)
        def _():
            for k2 in range(min(nbuf, n_steps)):
                pltpu.make_async_copy(
                    bufs.at[k2],
                    out_hbm.at[pl.ds(0, block_m), pl.ds(0, block_n)],
                    sems.at[k2],
                ).wait()

    partial = pl.pallas_call(
        mm_kernel,
        grid=(n_full, m_blocks),
        in_specs=[
            pl.BlockSpec((block_m, D), lambda n, m: (m, 0)),
            pl.BlockSpec((D, block_n), lambda n, m: (0, n)),
            pl.BlockSpec((1, block_n), lambda n, m: (0, n)),
        ],
        out_specs=pl.BlockSpec(memory_space=pl.ANY),
        out_shape=jax.ShapeDtypeStruct((B, V), jnp.float32),
        scratch_shapes=[
            pltpu.VMEM((nbuf, block_m, block_n), jnp.float32),
            pltpu.SemaphoreType.DMA((nbuf,)),
        ],
    )(emb, W, b2)

    if n_full * block_n == V:
        return partial

    def edge_kernel(emb_ref, w_ref, b_ref, full_ref, out_ref):
        del full_ref
        out_ref[...] = (
            jnp.dot(
                emb_ref[...],
                w_ref[...].astype(jnp.bfloat16),
                preferred_element_type=jnp.float32,
            )
            + b_ref[...]
        )

    return pl.pallas_call(
        edge_kernel,
        grid=(m_blocks,),
        in_specs=[
            pl.BlockSpec((block_m, D), lambda m: (m, 0)),
            pl.BlockSpec((D, block_n), lambda m: (0, n_full)),
            pl.BlockSpec((1, block_n), lambda m: (0, n_full)),
            pl.BlockSpec(memory_space=pl.ANY),
        ],
        out_specs=pl.BlockSpec((block_m, block_n), lambda m: (m, n_full)),
        out_shape=jax.ShapeDtypeStruct((B, V), jnp.float32),
        input_output_aliases={3: 0},
    )(emb, W, b2, partial)


def _band_write_probe(B, V, band, nbuf):
    m_blocks = B // band

    def probe_kernel(out_hbm, buf, sems):
        m = pl.program_id(0)
        pltpu.make_async_copy(
            buf.at[0], out_hbm.at[pl.ds(m * band, band), :], sems.at[0]
        ).start()

        @pl.when(m == m_blocks - 1)
        def _():
            def drain(i, c):
                pltpu.make_async_copy(
                    buf.at[0], out_hbm.at[pl.ds(0, band), :], sems.at[0]
                ).wait()
                return c

            jax.lax.fori_loop(0, m_blocks, drain, 0)

    return pl.pallas_call(
        probe_kernel,
        grid=(m_blocks,),
        out_specs=pl.BlockSpec(memory_space=pl.ANY),
        out_shape=jax.ShapeDtypeStruct((B, V), jnp.float32),
        scratch_shapes=[
            pltpu.VMEM((nbuf, band, V), jnp.float32),
            pltpu.SemaphoreType.DMA((nbuf,)),
        ],
    )()


def kernel(inputs, table, W, b):
    del inputs, table
    return _band_write_probe(4096, W.shape[1], band=16, nbuf=4)


# speedup vs baseline: 3.3880x; 3.2060x over previous
import functools

import jax
import jax.numpy as jnp
from jax import lax
from jax.experimental import pallas as pl
from jax.experimental.pallas import tpu as pltpu
from jax.experimental.pallas import tpu_sc as plsc


def _sc_gather(inputs, table):
    B = inputs.shape[0]
    V, D = table.shape
    info = plsc.get_sparse_core_info()
    nw = info.num_cores * info.num_subcores
    b_per_w = B // nw
    mesh = plsc.VectorSubcoreMesh(core_axis_name="c", subcore_axis_name="s")

    @functools.partial(
        pl.kernel,
        mesh=mesh,
        out_type=jax.ShapeDtypeStruct((B, D), jnp.float32),
        scratch_types=[
            pltpu.VMEM((b_per_w,), jnp.int32),
            pltpu.VMEM((b_per_w, D), jnp.float32),
            pltpu.SemaphoreType.DMA,
        ],
    )
    def gather_kernel(idx_hbm, table_hbm, out_hbm, idx_v, rows_v, sem):
        wid = lax.axis_index("s") * info.num_cores + lax.axis_index("c")
        base = wid * b_per_w
        pltpu.sync_copy(idx_hbm.at[pl.ds(base, b_per_w)], idx_v)
        pltpu.async_copy(table_hbm.at[idx_v], rows_v, sem).wait()
        pltpu.sync_copy(rows_v, out_hbm.at[pl.ds(base, b_per_w)])

    return gather_kernel(inputs, table)


def _tc_project_t(embT, Wt, b, block_n, nbuf):
    D, B = embT.shape
    V = Wt.shape[0]
    n_full = V // block_n
    rem = V - n_full * block_n
    grid = n_full + (1 if rem else 0)
    last = grid - 1
    last_rows = rem if rem else block_n
    bT = b.reshape(V, 1)

    def mm_kernel(embT_ref, wt_ref, b_ref, out_hbm, bufs, sems):
        j = pl.program_id(0)
        slot = jax.lax.rem(j, nbuf)
        acc = (
            jnp.dot(
                wt_ref[...].astype(jnp.bfloat16),
                embT_ref[...],
                preferred_element_type=jnp.float32,
            )
            + b_ref[...]
        )
        for k in range(nbuf):
            @pl.when(jnp.logical_and(slot == k, j >= nbuf))
            def _():
                pltpu.make_async_copy(
                    bufs.at[k], out_hbm.at[pl.ds(0, block_n), :], sems.at[k]
                ).wait()

            @pl.when(slot == k)
            def _():
                bufs[k] = acc

            @pl.when(jnp.logical_and(slot == k, j != last))
            def _():
                pltpu.make_async_copy(
                    bufs.at[k],
                    out_hbm.at[pl.ds(j * block_n, block_n), :],
                    sems.at[k],
                ).start()

        @pl.when(j == last)
        def _():
            lk = last % nbuf
            pltpu.make_async_copy(
                bufs.at[lk].at[pl.ds(0, last_rows), :],
                out_hbm.at[pl.ds(last * block_n, last_rows), :],
                sems.at[lk],
            ).start()
            for k2 in range(min(nbuf, grid)):
                rows = last_rows if k2 == last % nbuf else block_n
                pltpu.make_async_copy(
                    bufs.at[k2].at[pl.ds(0, rows), :],
                    out_hbm.at[pl.ds(0, rows), :],
                    sems.at[k2],
                ).wait()

    return pl.pallas_call(
        mm_kernel,
        grid=(grid,),
        in_specs=[
            pl.BlockSpec((D, B), lambda j: (0, 0)),
            pl.BlockSpec((block_n, D), lambda j: (j, 0)),
            pl.BlockSpec((block_n, 1), lambda j: (j, 0)),
        ],
        out_specs=pl.BlockSpec(memory_space=pl.ANY),
        out_shape=jax.ShapeDtypeStruct((V, B), jnp.float32),
        scratch_shapes=[
            pltpu.VMEM((nbuf, block_n, B), jnp.float32),
            pltpu.SemaphoreType.DMA((nbuf,)),
        ],
    )(embT, Wt, bT)


def kernel(inputs, table, W, b):
    emb = _sc_gather(inputs, table)
    embT = emb.T.astype(jnp.bfloat16)
    Wt = W.T
    outT = _tc_project_t(embT, Wt, b, block_n=512, nbuf=4)
    return outT.T


# speedup vs baseline: 3.7553x; 1.1084x over previous
import functools

import jax
import jax.numpy as jnp
from jax import lax
from jax.experimental import pallas as pl
from jax.experimental.pallas import tpu as pltpu
from jax.experimental.pallas import tpu_sc as plsc


def _sc_gather(inputs, table):
    B = inputs.shape[0]
    V, D = table.shape
    info = plsc.get_sparse_core_info()
    nw = info.num_cores * info.num_subcores
    b_per_w = B // nw
    mesh = plsc.VectorSubcoreMesh(core_axis_name="c", subcore_axis_name="s")

    @functools.partial(
        pl.kernel,
        mesh=mesh,
        out_type=jax.ShapeDtypeStruct((B, D), jnp.float32),
        scratch_types=[
            pltpu.VMEM((b_per_w,), jnp.int32),
            pltpu.VMEM((b_per_w, D), jnp.float32),
            pltpu.SemaphoreType.DMA,
        ],
    )
    def gather_kernel(idx_hbm, table_hbm, out_hbm, idx_v, rows_v, sem):
        wid = lax.axis_index("s") * info.num_cores + lax.axis_index("c")
        base = wid * b_per_w
        pltpu.sync_copy(idx_hbm.at[pl.ds(base, b_per_w)], idx_v)
        pltpu.async_copy(table_hbm.at[idx_v], rows_v, sem).wait()
        pltpu.sync_copy(rows_v, out_hbm.at[pl.ds(base, b_per_w)])

    return gather_kernel(inputs, table)


def _tc_project_t(embT, Wt, b, block_n):
    D, B = embT.shape
    V = Wt.shape[0]
    grid = pl.cdiv(V, block_n)
    b2 = b.reshape(1, V)

    def mm_kernel(embT_ref, wt_ref, b_ref, out_ref):
        out_ref[...] = (
            jnp.dot(
                wt_ref[...].astype(jnp.bfloat16),
                embT_ref[...],
                preferred_element_type=jnp.float32,
            )
            + jnp.transpose(b_ref[...])
        )

    return pl.pallas_call(
        mm_kernel,
        grid=(grid,),
        in_specs=[
            pl.BlockSpec((D, B), lambda j: (0, 0)),
            pl.BlockSpec((block_n, D), lambda j: (j, 0)),
            pl.BlockSpec((1, block_n), lambda j: (0, j)),
        ],
        out_specs=pl.BlockSpec((block_n, B), lambda j: (j, 0)),
        out_shape=jax.ShapeDtypeStruct((V, B), jnp.float32),
    )(embT, Wt, b2)


def kernel(inputs, table, W, b):
    emb = _sc_gather(inputs, table)
    embT = emb.T.astype(jnp.bfloat16)
    Wt = W.T
    outT = _tc_project_t(embT, Wt, b, block_n=512)
    return outT.T


# speedup vs baseline: 3.8155x; 1.0160x over previous
import functools

import jax
import jax.numpy as jnp
from jax import lax
from jax.experimental import pallas as pl
from jax.experimental.pallas import tpu as pltpu
from jax.experimental.pallas import tpu_sc as plsc


def _sc_gather(inputs, table):
    B = inputs.shape[0]
    V, D = table.shape
    info = plsc.get_sparse_core_info()
    nw = info.num_cores * info.num_subcores
    b_per_w = B // nw
    mesh = plsc.VectorSubcoreMesh(core_axis_name="c", subcore_axis_name="s")

    @functools.partial(
        pl.kernel,
        mesh=mesh,
        out_type=jax.ShapeDtypeStruct((B, D), jnp.float32),
        scratch_types=[
            pltpu.VMEM((b_per_w,), jnp.int32),
            pltpu.VMEM((b_per_w, D), jnp.float32),
            pltpu.SemaphoreType.DMA,
        ],
    )
    def gather_kernel(idx_hbm, table_hbm, out_hbm, idx_v, rows_v, sem):
        wid = lax.axis_index("s") * info.num_cores + lax.axis_index("c")
        base = wid * b_per_w
        pltpu.sync_copy(idx_hbm.at[pl.ds(base, b_per_w)], idx_v)
        pltpu.async_copy(table_hbm.at[idx_v], rows_v, sem).wait()
        pltpu.sync_copy(rows_v, out_hbm.at[pl.ds(base, b_per_w)])

    return gather_kernel(inputs, table)


def _tc_project_t(emb, Wt, b, block_n):
    B, D = emb.shape
    V = Wt.shape[0]
    grid = pl.cdiv(V, block_n)
    b2 = b.reshape(1, V)

    def mm_kernel(emb_ref, wt_ref, b_ref, out_ref):
        out_ref[...] = (
            lax.dot_general(
                wt_ref[...].astype(jnp.bfloat16),
                emb_ref[...].astype(jnp.bfloat16),
                (((1,), (1,)), ((), ())),
                preferred_element_type=jnp.float32,
            )
            + jnp.transpose(b_ref[...])
        )

    return pl.pallas_call(
        mm_kernel,
        grid=(grid,),
        in_specs=[
            pl.BlockSpec((B, D), lambda j: (0, 0)),
            pl.BlockSpec((block_n, D), lambda j: (j, 0)),
            pl.BlockSpec((1, block_n), lambda j: (0, j)),
        ],
        out_specs=pl.BlockSpec((block_n, B), lambda j: (j, 0)),
        out_shape=jax.ShapeDtypeStruct((V, B), jnp.float32),
    )(emb, Wt, b2)


def kernel(inputs, table, W, b):
    emb = _sc_gather(inputs, table)
    Wt = W.T
    outT = _tc_project_t(emb, Wt, b, block_n=1024)
    return outT.T
